# Initial kernel scaffold; baseline (speedup 1.0000x reference)
import jax
import jax.numpy as jnp
from jax.experimental import pallas as pl


def kernel(x, Wq, Wk, Wv, Wo):
    raise NotImplementedError("write your pallas kernel here")


# speedup vs baseline: 2.7374x; 2.7374x over previous
import jax
import jax.numpy as jnp
from jax.experimental import pallas as pl
from jax.experimental.pallas import tpu as pltpu

R = 1024
NH = 16
DH = 64
SCALE = 0.125


def _proj_body(xq_ref, xkv_ref, wq_ref, wk_ref, wv_ref, q_ref, k_ref, v_ref):
    xq = xq_ref[0]
    xkv = xkv_ref[0]
    q_ref[0] = jnp.dot(xq, wq_ref[...], preferred_element_type=jnp.float32)
    k_ref[0] = jnp.dot(xkv, wk_ref[...], preferred_element_type=jnp.float32)
    v_ref[0] = jnp.dot(xkv, wv_ref[...], preferred_element_type=jnp.float32)


def _attn_body(q_ref, k_ref, v_ref, acc_ref):
    c = pl.program_id(1)
    coef = jnp.where(c == 0, 2.0 / 3.0, jnp.where(c == 1, 1.0 / 3.0, 1.0))
    accumulate = c == 1
    for h in range(NH):
        sl = slice(h * DH, (h + 1) * DH)
        q = q_ref[0, :, sl]
        k = k_ref[0, :, sl]
        v = v_ref[0, :, sl]
        s = jax.lax.dot_general(
            q, k, (((1,), (1,)), ((), ())),
            preferred_element_type=jnp.float32) * SCALE
        m = jnp.max(s, axis=1, keepdims=True)
        p = jnp.exp(s - m)
        l = jnp.sum(p, axis=1, keepdims=True)
        o = jnp.dot(p, v, preferred_element_type=jnp.float32)
        o = o * (coef / l)

        @pl.when(accumulate)
        def _():
            acc_ref[0, :, sl] += o

        @pl.when(jnp.logical_not(accumulate))
        def _():
            acc_ref[0, :, sl] = o


def _out_body(acc_ref, wo_ref, out_ref):
    out_ref[0] = jnp.dot(acc_ref[0], wo_ref[...],
                         preferred_element_type=jnp.float32)


def kernel(x, Wq, Wk, Wv, Wo):
    B, N, D = x.shape
    f32 = jnp.float32
    RP = 512
    q, k, v = pl.pallas_call(
        _proj_body,
        grid=(B, (2 * R) // RP),
        in_specs=[
            pl.BlockSpec((1, RP, D), lambda b, j: (b, j, 0)),
            pl.BlockSpec((1, RP, D), lambda b, j: (b, j + R // RP, 0)),
            pl.BlockSpec((D, D), lambda b, j: (0, 0)),
            pl.BlockSpec((D, D), lambda b, j: (0, 0)),
            pl.BlockSpec((D, D), lambda b, j: (0, 0)),
        ],
        out_specs=[
            pl.BlockSpec((1, RP, D), lambda b, j: (b, j, 0)),
            pl.BlockSpec((1, RP, D), lambda b, j: (b, j, 0)),
            pl.BlockSpec((1, RP, D), lambda b, j: (b, j, 0)),
        ],
        out_shape=[jax.ShapeDtypeStruct((B, 2 * R, D), f32)] * 3,
        compiler_params=pltpu.CompilerParams(
            dimension_semantics=("parallel", "arbitrary")),
    )(x, x, Wq, Wk, Wv)

    acc = pl.pallas_call(
        _attn_body,
        grid=(B, 3),
        in_specs=[
            pl.BlockSpec((1, R, D), lambda b, c: (b, c // 2, 0)),
            pl.BlockSpec((1, R, D), lambda b, c: (b, (c + 1) // 2, 0)),
            pl.BlockSpec((1, R, D), lambda b, c: (b, (c + 1) // 2, 0)),
        ],
        out_specs=pl.BlockSpec((1, R, D), lambda b, c: (b, c // 2, 0)),
        out_shape=jax.ShapeDtypeStruct((B, 2 * R, D), f32),
        compiler_params=pltpu.CompilerParams(
            dimension_semantics=("arbitrary", "arbitrary")),
    )(q, k, v)

    out01 = pl.pallas_call(
        _out_body,
        grid=(B, 2),
        in_specs=[
            pl.BlockSpec((1, R, D), lambda b, j: (b, j, 0)),
            pl.BlockSpec((D, D), lambda b, j: (0, 0)),
        ],
        out_specs=pl.BlockSpec((1, R, D), lambda b, j: (b, j, 0)),
        out_shape=jax.ShapeDtypeStruct((B, 2 * R, D), f32),
        compiler_params=pltpu.CompilerParams(
            dimension_semantics=("parallel", "arbitrary")),
    )(acc, Wo)

    return jnp.concatenate([out01, x[:, 2 * R:, :]], axis=1)
